# baseline (speedup 1.0000x reference)
import functools

import jax
import jax.numpy as jnp
from jax import lax
from jax.experimental import pallas as pl
from jax.experimental.pallas import tpu as pltpu
from jax.experimental.pallas import tpu_sc as plsc

N = 10000
E = 320000
D = 128
NC = 2
NS = 16
NW = NC * NS
NB = 80
E_PAD = NW * NB * 128
ICH = 8
NPAD = NB * 128
RSHARE = 624

_mesh = plsc.VectorSubcoreMesh(core_axis_name="c", subcore_axis_name="s")


def _wid():
    return lax.axis_index("s") * NC + lax.axis_index("c")


@functools.partial(
    pl.kernel,
    out_type=jax.ShapeDtypeStruct((NW, NB, 128), jnp.float32),
    mesh=_mesh,
    compiler_params=pltpu.CompilerParams(needs_layout_passes=False),
    scratch_types=[
        pltpu.VMEM((NB, 128), jnp.float32),
        pltpu.VMEM((NB, 128), jnp.int32),
        pltpu.VMEM((NB, 128), jnp.int32),
        pltpu.VMEM((NB, 128), jnp.float32),
    ],
)
def _deg_kernel(src_h, dst_h, w_h, out_h, deg_v, src_v, dst_v, w_v):
    wid = _wid()
    zero16 = jnp.zeros((16,), jnp.float32)
    pltpu.sync_copy(src_h.at[wid], src_v)
    pltpu.sync_copy(dst_h.at[wid], dst_v)
    pltpu.sync_copy(w_h.at[wid], w_v)

    def zbody(i, _):
        for k in range(8):
            deg_v[i, pl.ds(k * 16, 16)] = zero16
        return 0

    lax.fori_loop(0, NB, zbody, 0)

    def bbody(b, _):
        for k in range(8):
            sl = pl.ds(k * 16, 16)
            r = src_v[b, sl]
            cc = dst_v[b, sl]
            w = w_v[b, sl]
            wm = jnp.where(r == cc, 0.0, w)
            plsc.addupdate_scatter(
                deg_v,
                [lax.shift_right_logical(r, 7), lax.bitwise_and(r, 127)],
                wm)
        return 0

    lax.fori_loop(0, NB, bbody, 0)
    pltpu.sync_copy(deg_v, out_h.at[wid])


@functools.partial(
    pl.kernel,
    out_type=jax.ShapeDtypeStruct((NW, NB, 128), jnp.float32),
    mesh=_mesh,
    compiler_params=pltpu.CompilerParams(needs_layout_passes=False),
    scratch_types=[
        pltpu.VMEM((NPAD,), jnp.float32),
        pltpu.VMEM((NB, 128), jnp.int32),
        pltpu.VMEM((NB, 128), jnp.int32),
        pltpu.VMEM((NB, 128), jnp.float32),
        pltpu.VMEM((NB, 128), jnp.float32),
    ],
)
def _norm_kernel(src_h, dst_h, w_h, dis_h, out_h, dis_v, src_v, dst_v, w_v, nrm_v):
    wid = _wid()
    pltpu.sync_copy(dis_h, dis_v)
    pltpu.sync_copy(src_h.at[wid], src_v)
    pltpu.sync_copy(dst_h.at[wid], dst_v)
    pltpu.sync_copy(w_h.at[wid], w_v)

    def bbody(b, _):
        for k in range(8):
            sl = pl.ds(k * 16, 16)
            r = src_v[b, sl]
            cc = dst_v[b, sl]
            w = w_v[b, sl]
            wm = jnp.where(r == cc, 0.0, w)
            dr = plsc.load_gather(dis_v, [r])
            dc = plsc.load_gather(dis_v, [cc])
            nrm_v[b, sl] = -(dr * wm * dc)
        return 0

    lax.fori_loop(0, NB, bbody, 0)
    pltpu.sync_copy(nrm_v, out_h.at[wid])


NCH = NB // ICH
TB = E_PAD // 128
CNT0 = 80
CNT1 = (TB // NS) - CNT0


@functools.partial(
    pl.kernel,
    out_type=jax.ShapeDtypeStruct((NC, N, D), jnp.float32),
    mesh=_mesh,
    compiler_params=pltpu.CompilerParams(needs_layout_passes=False),
    scratch_types=[
        pltpu.VMEM((2, ICH, 128), jnp.int32),
        pltpu.VMEM((2, ICH, 128), jnp.int32),
        pltpu.VMEM((2, ICH, 128), jnp.float32),
        pltpu.VMEM((2, 128, D), jnp.float32),
        pltpu.VMEM_SHARED((N, D), jnp.float32),
        pltpu.SemaphoreType.DMA,
        pltpu.SemaphoreType.DMA,
        pltpu.SemaphoreType.DMA,
    ],
)
def _prop_kernel(x_h, src_h, dst_h, nrm_h, out_h, src_v, dst_v, nrm_v, rows_v,
                 acc_s, semg, semi, sems):
    cid = lax.axis_index("c")
    sid = lax.axis_index("s")
    start = jnp.where(cid == 0, sid * CNT0, NS * CNT0 + sid * CNT1)
    ncz = jnp.where(cid == 0, CNT0 // ICH, CNT1 // ICH)
    zero16 = jnp.zeros((16,), jnp.float32)

    def zbody(i, _):
        for j in range(D // 16):
            rows_v[0, i, pl.ds(j * 16, 16)] = zero16
        return 0

    lax.fori_loop(0, 128, zbody, 0)
    zbase = sid * RSHARE
    for r in range(4):
        pltpu.sync_copy(rows_v.at[0], acc_s.at[pl.ds(zbase + r * 128, 128)])
    pltpu.sync_copy(rows_v.at[0, pl.ds(0, RSHARE - 512)],
                    acc_s.at[pl.ds(zbase + 512, RSHARE - 512)])

    @pl.when(sid == NS - 1)
    def _():
        pltpu.sync_copy(rows_v.at[0, pl.ds(0, N - NS * RSHARE)],
                        acc_s.at[pl.ds(NS * RSHARE, N - NS * RSHARE)])

    plsc.subcore_barrier()

    pltpu.sync_copy(src_h.at[pl.ds(start, ICH)], src_v.at[0])
    pltpu.sync_copy(dst_h.at[pl.ds(start, ICH)], dst_v.at[0])
    pltpu.sync_copy(nrm_h.at[pl.ds(start, ICH)], nrm_v.at[0])
    pltpu.async_copy(x_h.at[src_v.at[0, 0, pl.ds(0, 64)]],
                     rows_v.at[0, pl.ds(0, 64)], semg)
    pltpu.async_copy(x_h.at[src_v.at[0, 0, pl.ds(64, 64)]],
                     rows_v.at[0, pl.ds(64, 64)], semg)

    def chunk(c, _):
        par = lax.bitwise_and(c, 1)
        npar = 1 - par
        nbase = start + (c + 1) * ICH

        @pl.when(c > 0)
        def _():
            pltpu.make_async_copy(rows_v.at[1],
                                  acc_s.at[dst_v.at[npar, ICH - 1]],
                                  sems).wait()

        @pl.when(c < ncz - 1)
        def _():
            pltpu.async_copy(src_h.at[pl.ds(nbase, ICH)],
                             src_v.at[npar], semi)
            pltpu.async_copy(dst_h.at[pl.ds(nbase, ICH)],
                             dst_v.at[npar], semi)
            pltpu.async_copy(nrm_h.at[pl.ds(nbase, ICH)],
                             nrm_v.at[npar], semi)

        for b in range(ICH):
            buf = b % 2
            pltpu.make_async_copy(x_h.at[src_v.at[par, b, pl.ds(0, 64)]],
                                  rows_v.at[buf, pl.ds(0, 64)], semg).wait()
            pltpu.make_async_copy(x_h.at[src_v.at[par, b, pl.ds(64, 64)]],
                                  rows_v.at[buf, pl.ds(64, 64)], semg).wait()

            def gbody(g, _):
                nv = nrm_v[par, b, pl.ds(g * 16, 16)]
                for l in range(16):
                    ns = nv[l]
                    ri = g * 16 + l
                    for j in range(D // 16):
                        sl = pl.ds(j * 16, 16)
                        rows_v[buf, ri, sl] = rows_v[buf, ri, sl] * ns
                return 0

            lax.fori_loop(0, 8, gbody, 0)
            pltpu.async_copy(rows_v.at[buf], acc_s.at[dst_v.at[par, b]],
                             sems, add=True)
            if b > 0:
                pltpu.make_async_copy(rows_v.at[1 - buf],
                                      acc_s.at[dst_v.at[par, b - 1]],
                                      sems).wait()
            if b < ICH - 1:
                pltpu.async_copy(x_h.at[src_v.at[par, b + 1, pl.ds(0, 64)]],
                                 rows_v.at[1 - buf, pl.ds(0, 64)], semg)
                pltpu.async_copy(x_h.at[src_v.at[par, b + 1, pl.ds(64, 64)]],
                                 rows_v.at[1 - buf, pl.ds(64, 64)], semg)
            else:
                @pl.when(c < ncz - 1)
                def _():
                    pltpu.make_async_copy(src_h.at[pl.ds(nbase, ICH)],
                                          src_v.at[npar], semi).wait()
                    pltpu.make_async_copy(dst_h.at[pl.ds(nbase, ICH)],
                                          dst_v.at[npar], semi).wait()
                    pltpu.make_async_copy(nrm_h.at[pl.ds(nbase, ICH)],
                                          nrm_v.at[npar], semi).wait()
                    pltpu.async_copy(x_h.at[src_v.at[npar, 0, pl.ds(0, 64)]],
                                     rows_v.at[1 - buf, pl.ds(0, 64)], semg)
                    pltpu.async_copy(x_h.at[src_v.at[npar, 0, pl.ds(64, 64)]],
                                     rows_v.at[1 - buf, pl.ds(64, 64)], semg)
        return 0

    lax.fori_loop(0, ncz, chunk, 0)
    lastpar = lax.bitwise_and(ncz - 1, 1)
    pltpu.make_async_copy(rows_v.at[1],
                          acc_s.at[dst_v.at[lastpar, ICH - 1]],
                          sems).wait()
    plsc.subcore_barrier()
    obase = sid * RSHARE
    pltpu.sync_copy(acc_s.at[pl.ds(obase, RSHARE)],
                    out_h.at[cid, pl.ds(obase, RSHARE)])

    @pl.when(sid == NS - 1)
    def _():
        pltpu.sync_copy(acc_s.at[pl.ds(NS * RSHARE, N - NS * RSHARE)],
                        out_h.at[cid, pl.ds(NS * RSHARE, N - NS * RSHARE)])


BN = 1000
GRID = N // BN


def _dis_body(dp_ref, dis_ref):
    deg = jnp.sum(dp_ref[...], axis=0)
    safe = jnp.where(deg > 0.0, deg, 1.0)
    dis_ref[...] = jnp.where(deg > 0.0, lax.rsqrt(safe), 0.0)


_dis_tc = pl.pallas_call(
    _dis_body,
    grid=(5,),
    in_specs=[pl.BlockSpec((NW, 16, 128), lambda i: (0, i, 0))],
    out_specs=pl.BlockSpec((16, 128), lambda i: (i, 0)),
    out_shape=jax.ShapeDtypeStruct((NB, 128), jnp.float32),
)


def _mm_body(x_ref, w_ref, o_ref):
    o_ref[...] = jnp.dot(x_ref[...], w_ref[...],
                         preferred_element_type=jnp.float32)


_mm_tc = pl.pallas_call(
    _mm_body,
    grid=(GRID,),
    in_specs=[
        pl.BlockSpec((BN, D), lambda i: (i, 0)),
        pl.BlockSpec((D, D), lambda i: (0, 0)),
    ],
    out_specs=pl.BlockSpec((BN, D), lambda i: (i, 0)),
    out_shape=jax.ShapeDtypeStruct((N, D), jnp.float32),
)


def _comb1_body(p_ref, acc_ref, w_ref, tx_ref, out_ref):
    tx = p_ref[0] + p_ref[1]
    tx_ref[...] = tx
    out_ref[...] = acc_ref[...] + jnp.dot(
        tx, w_ref[...], preferred_element_type=jnp.float32)


_comb1_tc = pl.pallas_call(
    _comb1_body,
    grid=(GRID,),
    in_specs=[
        pl.BlockSpec((NC, BN, D), lambda i: (0, i, 0)),
        pl.BlockSpec((BN, D), lambda i: (i, 0)),
        pl.BlockSpec((D, D), lambda i: (0, 0)),
    ],
    out_specs=[
        pl.BlockSpec((BN, D), lambda i: (i, 0)),
        pl.BlockSpec((BN, D), lambda i: (i, 0)),
    ],
    out_shape=[
        jax.ShapeDtypeStruct((N, D), jnp.float32),
        jax.ShapeDtypeStruct((N, D), jnp.float32),
    ],
)


def _comb2_body(p_ref, x0_ref, acc_ref, w2_ref, b_ref, wn_ref, h_ref, hacc_ref):
    tx2 = 2.0 * (p_ref[0] + p_ref[1]) - x0_ref[...]
    h = acc_ref[...] + jnp.dot(tx2, w2_ref[...],
                               preferred_element_type=jnp.float32) + b_ref[...]
    h = jnp.maximum(h, 0.0)
    h_ref[...] = h
    hacc_ref[...] = jnp.dot(h, wn_ref[...], preferred_element_type=jnp.float32)


_comb2_tc = pl.pallas_call(
    _comb2_body,
    grid=(GRID,),
    in_specs=[
        pl.BlockSpec((NC, BN, D), lambda i: (0, i, 0)),
        pl.BlockSpec((BN, D), lambda i: (i, 0)),
        pl.BlockSpec((BN, D), lambda i: (i, 0)),
        pl.BlockSpec((D, D), lambda i: (0, 0)),
        pl.BlockSpec((1, D), lambda i: (0, 0)),
        pl.BlockSpec((D, D), lambda i: (0, 0)),
    ],
    out_specs=[
        pl.BlockSpec((BN, D), lambda i: (i, 0)),
        pl.BlockSpec((BN, D), lambda i: (i, 0)),
    ],
    out_shape=[
        jax.ShapeDtypeStruct((N, D), jnp.float32),
        jax.ShapeDtypeStruct((N, D), jnp.float32),
    ],
)


def _final_body(p_ref, x0_ref, acc_ref, w_ref, b_ref, o_ref):
    tx2 = 2.0 * (p_ref[0] + p_ref[1]) - x0_ref[...]
    o = acc_ref[...] + jnp.dot(tx2, w_ref[...],
                               preferred_element_type=jnp.float32) + b_ref[...]
    m = jnp.max(o, axis=1, keepdims=True)
    lse = m + jnp.log(jnp.sum(jnp.exp(o - m), axis=1, keepdims=True))
    o_ref[...] = o - lse


_final_tc = pl.pallas_call(
    _final_body,
    grid=(GRID,),
    in_specs=[
        pl.BlockSpec((NC, BN, D), lambda i: (0, i, 0)),
        pl.BlockSpec((BN, D), lambda i: (i, 0)),
        pl.BlockSpec((BN, D), lambda i: (i, 0)),
        pl.BlockSpec((D, D), lambda i: (0, 0)),
        pl.BlockSpec((1, D), lambda i: (0, 0)),
    ],
    out_specs=pl.BlockSpec((BN, D), lambda i: (i, 0)),
    out_shape=jax.ShapeDtypeStruct((N, D), jnp.float32),
)


def kernel(x, edge_index, edge_weight, W1, b1, W2, b2):
    pad = E_PAD - E
    src = jnp.concatenate(
        [edge_index[0], jnp.zeros((pad,), jnp.int32)]).reshape(NW, NB, 128)
    dst = jnp.concatenate(
        [edge_index[1], jnp.zeros((pad,), jnp.int32)]).reshape(NW, NB, 128)
    w = jnp.concatenate(
        [edge_weight, jnp.zeros((pad,), jnp.float32)]).reshape(NW, NB, 128)
    b1r = b1.reshape(1, D)
    b2r = b2.reshape(1, D)

    deg_parts = _deg_kernel(src, dst, w)
    dis = _dis_tc(deg_parts)
    nrm = _norm_kernel(src, dst, w, dis.reshape(NPAD))

    srcf = src.reshape(TB, 128)
    dstf = dst.reshape(TB, 128)
    nrmf = nrm.reshape(TB, 128)

    acc0 = _mm_tc(x, W1[0])
    p1 = _prop_kernel(x, srcf, dstf, nrmf)
    tx1, acc1 = _comb1_tc(p1, acc0, W1[1])
    p2 = _prop_kernel(tx1, srcf, dstf, nrmf)
    h, hacc = _comb2_tc(p2, x, acc1, W1[2], b1r, W2[0])
    p3 = _prop_kernel(h, srcf, dstf, nrmf)
    t1, acc2 = _comb1_tc(p3, hacc, W2[1])
    p4 = _prop_kernel(t1, srcf, dstf, nrmf)
    return _final_tc(p4, h, acc2, W2[2], b2r)


# speedup vs baseline: 1.0306x; 1.0306x over previous
import functools

import jax
import jax.numpy as jnp
from jax import lax
from jax.experimental import pallas as pl
from jax.experimental.pallas import tpu as pltpu
from jax.experimental.pallas import tpu_sc as plsc

N = 10000
E = 320000
D = 128
NC = 2
NS = 16
NW = NC * NS
NB = 80
E_PAD = NW * NB * 128
ICH = 8
NPAD = NB * 128
RSHARE = 624

_mesh = plsc.VectorSubcoreMesh(core_axis_name="c", subcore_axis_name="s")


def _wid():
    return lax.axis_index("s") * NC + lax.axis_index("c")


@functools.partial(
    pl.kernel,
    out_type=jax.ShapeDtypeStruct((NW, NB, 128), jnp.float32),
    mesh=_mesh,
    compiler_params=pltpu.CompilerParams(needs_layout_passes=False),
    scratch_types=[
        pltpu.VMEM((NB, 128), jnp.float32),
        pltpu.VMEM((NB, 128), jnp.int32),
        pltpu.VMEM((NB, 128), jnp.int32),
        pltpu.VMEM((NB, 128), jnp.float32),
    ],
)
def _deg_kernel(src_h, dst_h, w_h, out_h, deg_v, src_v, dst_v, w_v):
    wid = _wid()
    zero16 = jnp.zeros((16,), jnp.float32)
    pltpu.sync_copy(src_h.at[wid], src_v)
    pltpu.sync_copy(dst_h.at[wid], dst_v)
    pltpu.sync_copy(w_h.at[wid], w_v)

    def zbody(i, _):
        for k in range(8):
            deg_v[i, pl.ds(k * 16, 16)] = zero16
        return 0

    lax.fori_loop(0, NB, zbody, 0)

    def bbody(b, _):
        for k in range(8):
            sl = pl.ds(k * 16, 16)
            r = src_v[b, sl]
            cc = dst_v[b, sl]
            w = w_v[b, sl]
            wm = jnp.where(r == cc, 0.0, w)
            plsc.addupdate_scatter(
                deg_v,
                [lax.shift_right_logical(r, 7), lax.bitwise_and(r, 127)],
                wm)
        return 0

    lax.fori_loop(0, NB, bbody, 0)
    pltpu.sync_copy(deg_v, out_h.at[wid])


@functools.partial(
    pl.kernel,
    out_type=jax.ShapeDtypeStruct((NW, NB, 128), jnp.float32),
    mesh=_mesh,
    compiler_params=pltpu.CompilerParams(needs_layout_passes=False),
    scratch_types=[
        pltpu.VMEM((NPAD,), jnp.float32),
        pltpu.VMEM((NB, 128), jnp.int32),
        pltpu.VMEM((NB, 128), jnp.int32),
        pltpu.VMEM((NB, 128), jnp.float32),
        pltpu.VMEM((NB, 128), jnp.float32),
    ],
)
def _norm_kernel(src_h, dst_h, w_h, dis_h, out_h, dis_v, src_v, dst_v, w_v, nrm_v):
    wid = _wid()
    pltpu.sync_copy(dis_h, dis_v)
    pltpu.sync_copy(src_h.at[wid], src_v)
    pltpu.sync_copy(dst_h.at[wid], dst_v)
    pltpu.sync_copy(w_h.at[wid], w_v)

    def bbody(b, _):
        for k in range(8):
            sl = pl.ds(k * 16, 16)
            r = src_v[b, sl]
            cc = dst_v[b, sl]
            w = w_v[b, sl]
            wm = jnp.where(r == cc, 0.0, w)
            dr = plsc.load_gather(dis_v, [r])
            dc = plsc.load_gather(dis_v, [cc])
            nrm_v[b, sl] = -(dr * wm * dc)
        return 0

    lax.fori_loop(0, NB, bbody, 0)
    pltpu.sync_copy(nrm_v, out_h.at[wid])


NCH = NB // ICH
TB = E_PAD // 128
CNT0 = 80
CNT1 = (TB // NS) - CNT0


@functools.partial(
    pl.kernel,
    out_type=jax.ShapeDtypeStruct((NC, N, D), jnp.float32),
    mesh=_mesh,
    compiler_params=pltpu.CompilerParams(needs_layout_passes=False),
    scratch_types=[
        pltpu.VMEM((2, ICH, 128), jnp.int32),
        pltpu.VMEM((2, ICH, 128), jnp.int32),
        pltpu.VMEM((2, ICH, 128), jnp.float32),
        pltpu.VMEM((2, 128, D), jnp.float32),
        pltpu.VMEM_SHARED((N, D), jnp.float32),
        pltpu.SemaphoreType.DMA,
        pltpu.SemaphoreType.DMA,
        pltpu.SemaphoreType.DMA,
    ],
)
def _prop_kernel(x_h, src_h, dst_h, nrm_h, out_h, src_v, dst_v, nrm_v, rows_v,
                 acc_s, semg, semi, sems):
    cid = lax.axis_index("c")
    sid = lax.axis_index("s")
    start = jnp.where(cid == 0, sid * CNT0, NS * CNT0 + sid * CNT1)
    ncz = jnp.where(cid == 0, CNT0 // ICH, CNT1 // ICH)
    zero16 = jnp.zeros((16,), jnp.float32)

    def zbody(i, _):
        for j in range(D // 16):
            rows_v[0, i, pl.ds(j * 16, 16)] = zero16
        return 0

    lax.fori_loop(0, 128, zbody, 0)
    zbase = sid * RSHARE
    for r in range(4):
        pltpu.sync_copy(rows_v.at[0], acc_s.at[pl.ds(zbase + r * 128, 128)])
    pltpu.sync_copy(rows_v.at[0, pl.ds(0, RSHARE - 512)],
                    acc_s.at[pl.ds(zbase + 512, RSHARE - 512)])

    @pl.when(sid == NS - 1)
    def _():
        pltpu.sync_copy(rows_v.at[0, pl.ds(0, N - NS * RSHARE)],
                        acc_s.at[pl.ds(NS * RSHARE, N - NS * RSHARE)])

    plsc.subcore_barrier()

    pltpu.sync_copy(src_h.at[pl.ds(start, ICH)], src_v.at[0])
    pltpu.sync_copy(dst_h.at[pl.ds(start, ICH)], dst_v.at[0])
    pltpu.sync_copy(nrm_h.at[pl.ds(start, ICH)], nrm_v.at[0])
    pltpu.async_copy(x_h.at[src_v.at[0, 0, pl.ds(0, 64)]],
                     rows_v.at[0, pl.ds(0, 64)], semg)
    pltpu.async_copy(x_h.at[src_v.at[0, 0, pl.ds(64, 64)]],
                     rows_v.at[0, pl.ds(64, 64)], semg)

    def chunk(c, _):
        par = lax.bitwise_and(c, 1)
        npar = 1 - par
        nbase = start + (c + 1) * ICH

        @pl.when(c > 0)
        def _():
            pltpu.make_async_copy(rows_v.at[1],
                                  acc_s.at[dst_v.at[npar, ICH - 1]],
                                  sems).wait()

        @pl.when(c < ncz - 1)
        def _():
            pltpu.async_copy(src_h.at[pl.ds(nbase, ICH)],
                             src_v.at[npar], semi)
            pltpu.async_copy(dst_h.at[pl.ds(nbase, ICH)],
                             dst_v.at[npar], semi)
            pltpu.async_copy(nrm_h.at[pl.ds(nbase, ICH)],
                             nrm_v.at[npar], semi)

        for b in range(ICH):
            buf = b % 2

            def gbody(g, _):
                nv = nrm_v[par, b, pl.ds(g * 16, 16)]
                for l in range(16):
                    ns = nv[l]
                    ri = g * 16 + l
                    for j in range(D // 16):
                        sl = pl.ds(j * 16, 16)
                        rows_v[buf, ri, sl] = rows_v[buf, ri, sl] * ns
                return 0

            pltpu.make_async_copy(x_h.at[src_v.at[par, b, pl.ds(0, 64)]],
                                  rows_v.at[buf, pl.ds(0, 64)], semg).wait()
            lax.fori_loop(0, 4, gbody, 0)
            pltpu.make_async_copy(x_h.at[src_v.at[par, b, pl.ds(64, 64)]],
                                  rows_v.at[buf, pl.ds(64, 64)], semg).wait()
            lax.fori_loop(4, 8, gbody, 0)
            pltpu.async_copy(rows_v.at[buf], acc_s.at[dst_v.at[par, b]],
                             sems, add=True)
            if b > 0:
                pltpu.make_async_copy(rows_v.at[1 - buf],
                                      acc_s.at[dst_v.at[par, b - 1]],
                                      sems).wait()
            if b < ICH - 1:
                pltpu.async_copy(x_h.at[src_v.at[par, b + 1, pl.ds(0, 64)]],
                                 rows_v.at[1 - buf, pl.ds(0, 64)], semg)
                pltpu.async_copy(x_h.at[src_v.at[par, b + 1, pl.ds(64, 64)]],
                                 rows_v.at[1 - buf, pl.ds(64, 64)], semg)
            else:
                @pl.when(c < ncz - 1)
                def _():
                    pltpu.make_async_copy(src_h.at[pl.ds(nbase, ICH)],
                                          src_v.at[npar], semi).wait()
                    pltpu.make_async_copy(dst_h.at[pl.ds(nbase, ICH)],
                                          dst_v.at[npar], semi).wait()
                    pltpu.make_async_copy(nrm_h.at[pl.ds(nbase, ICH)],
                                          nrm_v.at[npar], semi).wait()
                    pltpu.async_copy(x_h.at[src_v.at[npar, 0, pl.ds(0, 64)]],
                                     rows_v.at[1 - buf, pl.ds(0, 64)], semg)
                    pltpu.async_copy(x_h.at[src_v.at[npar, 0, pl.ds(64, 64)]],
                                     rows_v.at[1 - buf, pl.ds(64, 64)], semg)
        return 0

    lax.fori_loop(0, ncz, chunk, 0)
    lastpar = lax.bitwise_and(ncz - 1, 1)
    pltpu.make_async_copy(rows_v.at[1],
                          acc_s.at[dst_v.at[lastpar, ICH - 1]],
                          sems).wait()
    plsc.subcore_barrier()
    obase = sid * RSHARE
    pltpu.sync_copy(acc_s.at[pl.ds(obase, RSHARE)],
                    out_h.at[cid, pl.ds(obase, RSHARE)])

    @pl.when(sid == NS - 1)
    def _():
        pltpu.sync_copy(acc_s.at[pl.ds(NS * RSHARE, N - NS * RSHARE)],
                        out_h.at[cid, pl.ds(NS * RSHARE, N - NS * RSHARE)])


BN = 1000
GRID = N // BN


def _dis_body(dp_ref, dis_ref):
    deg = jnp.sum(dp_ref[...], axis=0)
    safe = jnp.where(deg > 0.0, deg, 1.0)
    dis_ref[...] = jnp.where(deg > 0.0, lax.rsqrt(safe), 0.0)


_dis_tc = pl.pallas_call(
    _dis_body,
    grid=(5,),
    in_specs=[pl.BlockSpec((NW, 16, 128), lambda i: (0, i, 0))],
    out_specs=pl.BlockSpec((16, 128), lambda i: (i, 0)),
    out_shape=jax.ShapeDtypeStruct((NB, 128), jnp.float32),
)


def _mm_body(x_ref, w_ref, o_ref):
    o_ref[...] = jnp.dot(x_ref[...], w_ref[...],
                         preferred_element_type=jnp.float32)


_mm_tc = pl.pallas_call(
    _mm_body,
    grid=(GRID,),
    in_specs=[
        pl.BlockSpec((BN, D), lambda i: (i, 0)),
        pl.BlockSpec((D, D), lambda i: (0, 0)),
    ],
    out_specs=pl.BlockSpec((BN, D), lambda i: (i, 0)),
    out_shape=jax.ShapeDtypeStruct((N, D), jnp.float32),
)


def _comb1_body(p_ref, acc_ref, w_ref, tx_ref, out_ref):
    tx = p_ref[0] + p_ref[1]
    tx_ref[...] = tx
    out_ref[...] = acc_ref[...] + jnp.dot(
        tx, w_ref[...], preferred_element_type=jnp.float32)


_comb1_tc = pl.pallas_call(
    _comb1_body,
    grid=(GRID,),
    in_specs=[
        pl.BlockSpec((NC, BN, D), lambda i: (0, i, 0)),
        pl.BlockSpec((BN, D), lambda i: (i, 0)),
        pl.BlockSpec((D, D), lambda i: (0, 0)),
    ],
    out_specs=[
        pl.BlockSpec((BN, D), lambda i: (i, 0)),
        pl.BlockSpec((BN, D), lambda i: (i, 0)),
    ],
    out_shape=[
        jax.ShapeDtypeStruct((N, D), jnp.float32),
        jax.ShapeDtypeStruct((N, D), jnp.float32),
    ],
)


def _comb2_body(p_ref, x0_ref, acc_ref, w2_ref, b_ref, wn_ref, h_ref, hacc_ref):
    tx2 = 2.0 * (p_ref[0] + p_ref[1]) - x0_ref[...]
    h = acc_ref[...] + jnp.dot(tx2, w2_ref[...],
                               preferred_element_type=jnp.float32) + b_ref[...]
    h = jnp.maximum(h, 0.0)
    h_ref[...] = h
    hacc_ref[...] = jnp.dot(h, wn_ref[...], preferred_element_type=jnp.float32)


_comb2_tc = pl.pallas_call(
    _comb2_body,
    grid=(GRID,),
    in_specs=[
        pl.BlockSpec((NC, BN, D), lambda i: (0, i, 0)),
        pl.BlockSpec((BN, D), lambda i: (i, 0)),
        pl.BlockSpec((BN, D), lambda i: (i, 0)),
        pl.BlockSpec((D, D), lambda i: (0, 0)),
        pl.BlockSpec((1, D), lambda i: (0, 0)),
        pl.BlockSpec((D, D), lambda i: (0, 0)),
    ],
    out_specs=[
        pl.BlockSpec((BN, D), lambda i: (i, 0)),
        pl.BlockSpec((BN, D), lambda i: (i, 0)),
    ],
    out_shape=[
        jax.ShapeDtypeStruct((N, D), jnp.float32),
        jax.ShapeDtypeStruct((N, D), jnp.float32),
    ],
)


def _final_body(p_ref, x0_ref, acc_ref, w_ref, b_ref, o_ref):
    tx2 = 2.0 * (p_ref[0] + p_ref[1]) - x0_ref[...]
    o = acc_ref[...] + jnp.dot(tx2, w_ref[...],
                               preferred_element_type=jnp.float32) + b_ref[...]
    m = jnp.max(o, axis=1, keepdims=True)
    lse = m + jnp.log(jnp.sum(jnp.exp(o - m), axis=1, keepdims=True))
    o_ref[...] = o - lse


_final_tc = pl.pallas_call(
    _final_body,
    grid=(GRID,),
    in_specs=[
        pl.BlockSpec((NC, BN, D), lambda i: (0, i, 0)),
        pl.BlockSpec((BN, D), lambda i: (i, 0)),
        pl.BlockSpec((BN, D), lambda i: (i, 0)),
        pl.BlockSpec((D, D), lambda i: (0, 0)),
        pl.BlockSpec((1, D), lambda i: (0, 0)),
    ],
    out_specs=pl.BlockSpec((BN, D), lambda i: (i, 0)),
    out_shape=jax.ShapeDtypeStruct((N, D), jnp.float32),
)


def kernel(x, edge_index, edge_weight, W1, b1, W2, b2):
    pad = E_PAD - E
    src = jnp.concatenate(
        [edge_index[0], jnp.zeros((pad,), jnp.int32)]).reshape(NW, NB, 128)
    dst = jnp.concatenate(
        [edge_index[1], jnp.zeros((pad,), jnp.int32)]).reshape(NW, NB, 128)
    w = jnp.concatenate(
        [edge_weight, jnp.zeros((pad,), jnp.float32)]).reshape(NW, NB, 128)
    b1r = b1.reshape(1, D)
    b2r = b2.reshape(1, D)

    deg_parts = _deg_kernel(src, dst, w)
    dis = _dis_tc(deg_parts)
    nrm = _norm_kernel(src, dst, w, dis.reshape(NPAD))

    srcf = src.reshape(TB, 128)
    dstf = dst.reshape(TB, 128)
    nrmf = nrm.reshape(TB, 128)

    acc0 = _mm_tc(x, W1[0])
    p1 = _prop_kernel(x, srcf, dstf, nrmf)
    tx1, acc1 = _comb1_tc(p1, acc0, W1[1])
    p2 = _prop_kernel(tx1, srcf, dstf, nrmf)
    h, hacc = _comb2_tc(p2, x, acc1, W1[2], b1r, W2[0])
    p3 = _prop_kernel(h, srcf, dstf, nrmf)
    t1, acc2 = _comb1_tc(p3, hacc, W2[1])
    p4 = _prop_kernel(t1, srcf, dstf, nrmf)
    return _final_tc(p4, h, acc2, W2[2], b2r)


# speedup vs baseline: 1.0554x; 1.0241x over previous
import functools

import jax
import jax.numpy as jnp
from jax import lax
from jax.experimental import pallas as pl
from jax.experimental.pallas import tpu as pltpu
from jax.experimental.pallas import tpu_sc as plsc

N = 10000
E = 320000
D = 128
NC = 2
NS = 16
NW = NC * NS
NB = 80
E_PAD = NW * NB * 128
ICH = 8
NPAD = NB * 128
RSHARE = 624

_mesh = plsc.VectorSubcoreMesh(core_axis_name="c", subcore_axis_name="s")


def _wid():
    return lax.axis_index("s") * NC + lax.axis_index("c")


@functools.partial(
    pl.kernel,
    out_type=jax.ShapeDtypeStruct((NW, NB, 128), jnp.float32),
    mesh=_mesh,
    compiler_params=pltpu.CompilerParams(needs_layout_passes=False),
    scratch_types=[
        pltpu.VMEM((NB, 128), jnp.float32),
        pltpu.VMEM((NB, 128), jnp.int32),
        pltpu.VMEM((NB, 128), jnp.int32),
        pltpu.VMEM((NB, 128), jnp.float32),
    ],
)
def _deg_kernel(src_h, dst_h, w_h, out_h, deg_v, src_v, dst_v, w_v):
    wid = _wid()
    zero16 = jnp.zeros((16,), jnp.float32)
    pltpu.sync_copy(src_h.at[wid], src_v)
    pltpu.sync_copy(dst_h.at[wid], dst_v)
    pltpu.sync_copy(w_h.at[wid], w_v)

    def zbody(i, _):
        for k in range(8):
            deg_v[i, pl.ds(k * 16, 16)] = zero16
        return 0

    lax.fori_loop(0, NB, zbody, 0)

    def bbody(b, _):
        for k in range(8):
            sl = pl.ds(k * 16, 16)
            r = src_v[b, sl]
            cc = dst_v[b, sl]
            w = w_v[b, sl]
            wm = jnp.where(r == cc, 0.0, w)
            plsc.addupdate_scatter(
                deg_v,
                [lax.shift_right_logical(r, 7), lax.bitwise_and(r, 127)],
                wm)
        return 0

    lax.fori_loop(0, NB, bbody, 0)
    pltpu.sync_copy(deg_v, out_h.at[wid])


@functools.partial(
    pl.kernel,
    out_type=jax.ShapeDtypeStruct((NW, NB, 128), jnp.float32),
    mesh=_mesh,
    compiler_params=pltpu.CompilerParams(needs_layout_passes=False),
    scratch_types=[
        pltpu.VMEM((NPAD,), jnp.float32),
        pltpu.VMEM((NB, 128), jnp.int32),
        pltpu.VMEM((NB, 128), jnp.int32),
        pltpu.VMEM((NB, 128), jnp.float32),
        pltpu.VMEM((NB, 128), jnp.float32),
    ],
)
def _norm_kernel(src_h, dst_h, w_h, dis_h, out_h, dis_v, src_v, dst_v, w_v, nrm_v):
    wid = _wid()
    pltpu.sync_copy(dis_h, dis_v)
    pltpu.sync_copy(src_h.at[wid], src_v)
    pltpu.sync_copy(dst_h.at[wid], dst_v)
    pltpu.sync_copy(w_h.at[wid], w_v)

    def bbody(b, _):
        for k in range(8):
            sl = pl.ds(k * 16, 16)
            r = src_v[b, sl]
            cc = dst_v[b, sl]
            w = w_v[b, sl]
            wm = jnp.where(r == cc, 0.0, w)
            dr = plsc.load_gather(dis_v, [r])
            dc = plsc.load_gather(dis_v, [cc])
            nrm_v[b, sl] = -(dr * wm * dc)
        return 0

    lax.fori_loop(0, NB, bbody, 0)
    pltpu.sync_copy(nrm_v, out_h.at[wid])


NCH = NB // ICH
TB = E_PAD // 128
CNT0 = 80
CNT1 = (TB // NS) - CNT0


@functools.partial(
    pl.kernel,
    out_type=jax.ShapeDtypeStruct((NC, N, D), jnp.float32),
    mesh=_mesh,
    compiler_params=pltpu.CompilerParams(needs_layout_passes=False),
    scratch_types=[
        pltpu.VMEM((2, ICH, 128), jnp.int32),
        pltpu.VMEM((2, ICH, 128), jnp.int32),
        pltpu.VMEM((2, ICH, 128), jnp.float32),
        pltpu.VMEM((2, 128, D), jnp.float32),
        pltpu.VMEM_SHARED((N, D), jnp.float32),
        pltpu.SemaphoreType.DMA,
        pltpu.SemaphoreType.DMA,
        pltpu.SemaphoreType.DMA,
    ],
)
def _prop_kernel(x_h, src_h, dst_h, nrm_h, out_h, src_v, dst_v, nrm_v, rows_v,
                 acc_s, semg, semi, sems):
    cid = lax.axis_index("c")
    sid = lax.axis_index("s")
    start = jnp.where(cid == 0, sid * CNT0, NS * CNT0 + sid * CNT1)
    ncz = jnp.where(cid == 0, CNT0 // ICH, CNT1 // ICH)
    zero16 = jnp.zeros((16,), jnp.float32)

    def zbody(i, _):
        for j in range(D // 16):
            rows_v[0, i, pl.ds(j * 16, 16)] = zero16
        return 0

    lax.fori_loop(0, 128, zbody, 0)
    zbase = sid * RSHARE
    for r in range(4):
        pltpu.sync_copy(rows_v.at[0], acc_s.at[pl.ds(zbase + r * 128, 128)])
    pltpu.sync_copy(rows_v.at[0, pl.ds(0, RSHARE - 512)],
                    acc_s.at[pl.ds(zbase + 512, RSHARE - 512)])

    @pl.when(sid == NS - 1)
    def _():
        pltpu.sync_copy(rows_v.at[0, pl.ds(0, N - NS * RSHARE)],
                        acc_s.at[pl.ds(NS * RSHARE, N - NS * RSHARE)])

    plsc.subcore_barrier()

    pltpu.sync_copy(src_h.at[pl.ds(start, ICH)], src_v.at[0])
    pltpu.sync_copy(dst_h.at[pl.ds(start, ICH)], dst_v.at[0])
    pltpu.sync_copy(nrm_h.at[pl.ds(start, ICH)], nrm_v.at[0])
    for q in range(4):
        pltpu.async_copy(x_h.at[src_v.at[0, 0, pl.ds(q * 32, 32)]],
                         rows_v.at[0, pl.ds(q * 32, 32)], semg)

    def chunk(c, _):
        par = lax.bitwise_and(c, 1)
        npar = 1 - par
        nbase = start + (c + 1) * ICH

        @pl.when(c > 0)
        def _():
            pltpu.make_async_copy(rows_v.at[1],
                                  acc_s.at[dst_v.at[npar, ICH - 1]],
                                  sems).wait()

        @pl.when(c < ncz - 1)
        def _():
            pltpu.async_copy(src_h.at[pl.ds(nbase, ICH)],
                             src_v.at[npar], semi)
            pltpu.async_copy(dst_h.at[pl.ds(nbase, ICH)],
                             dst_v.at[npar], semi)
            pltpu.async_copy(nrm_h.at[pl.ds(nbase, ICH)],
                             nrm_v.at[npar], semi)

        for b in range(ICH):
            buf = b % 2

            def gbody(g, _):
                nv = nrm_v[par, b, pl.ds(g * 16, 16)]
                for l in range(16):
                    ns = nv[l]
                    ri = g * 16 + l
                    for j in range(D // 16):
                        sl = pl.ds(j * 16, 16)
                        rows_v[buf, ri, sl] = rows_v[buf, ri, sl] * ns
                return 0

            for q in range(4):
                pltpu.make_async_copy(
                    x_h.at[src_v.at[par, b, pl.ds(q * 32, 32)]],
                    rows_v.at[buf, pl.ds(q * 32, 32)], semg).wait()
                lax.fori_loop(q * 2, q * 2 + 2, gbody, 0)
            pltpu.async_copy(rows_v.at[buf], acc_s.at[dst_v.at[par, b]],
                             sems, add=True)
            if b > 0:
                pltpu.make_async_copy(rows_v.at[1 - buf],
                                      acc_s.at[dst_v.at[par, b - 1]],
                                      sems).wait()
            if b < ICH - 1:
                for q in range(4):
                    pltpu.async_copy(
                        x_h.at[src_v.at[par, b + 1, pl.ds(q * 32, 32)]],
                        rows_v.at[1 - buf, pl.ds(q * 32, 32)], semg)
            else:
                @pl.when(c < ncz - 1)
                def _():
                    pltpu.make_async_copy(src_h.at[pl.ds(nbase, ICH)],
                                          src_v.at[npar], semi).wait()
                    pltpu.make_async_copy(dst_h.at[pl.ds(nbase, ICH)],
                                          dst_v.at[npar], semi).wait()
                    pltpu.make_async_copy(nrm_h.at[pl.ds(nbase, ICH)],
                                          nrm_v.at[npar], semi).wait()
                    for q in range(4):
                        pltpu.async_copy(
                            x_h.at[src_v.at[npar, 0, pl.ds(q * 32, 32)]],
                            rows_v.at[1 - buf, pl.ds(q * 32, 32)], semg)
        return 0

    lax.fori_loop(0, ncz, chunk, 0)
    lastpar = lax.bitwise_and(ncz - 1, 1)
    pltpu.make_async_copy(rows_v.at[1],
                          acc_s.at[dst_v.at[lastpar, ICH - 1]],
                          sems).wait()
    plsc.subcore_barrier()
    obase = sid * RSHARE
    pltpu.sync_copy(acc_s.at[pl.ds(obase, RSHARE)],
                    out_h.at[cid, pl.ds(obase, RSHARE)])

    @pl.when(sid == NS - 1)
    def _():
        pltpu.sync_copy(acc_s.at[pl.ds(NS * RSHARE, N - NS * RSHARE)],
                        out_h.at[cid, pl.ds(NS * RSHARE, N - NS * RSHARE)])


BN = 1000
GRID = N // BN


def _dis_body(dp_ref, dis_ref):
    deg = jnp.sum(dp_ref[...], axis=0)
    safe = jnp.where(deg > 0.0, deg, 1.0)
    dis_ref[...] = jnp.where(deg > 0.0, lax.rsqrt(safe), 0.0)


_dis_tc = pl.pallas_call(
    _dis_body,
    grid=(5,),
    in_specs=[pl.BlockSpec((NW, 16, 128), lambda i: (0, i, 0))],
    out_specs=pl.BlockSpec((16, 128), lambda i: (i, 0)),
    out_shape=jax.ShapeDtypeStruct((NB, 128), jnp.float32),
)


def _mm_body(x_ref, w_ref, o_ref):
    o_ref[...] = jnp.dot(x_ref[...], w_ref[...],
                         preferred_element_type=jnp.float32)


_mm_tc = pl.pallas_call(
    _mm_body,
    grid=(GRID,),
    in_specs=[
        pl.BlockSpec((BN, D), lambda i: (i, 0)),
        pl.BlockSpec((D, D), lambda i: (0, 0)),
    ],
    out_specs=pl.BlockSpec((BN, D), lambda i: (i, 0)),
    out_shape=jax.ShapeDtypeStruct((N, D), jnp.float32),
)


def _comb1_body(p_ref, acc_ref, w_ref, tx_ref, out_ref):
    tx = p_ref[0] + p_ref[1]
    tx_ref[...] = tx
    out_ref[...] = acc_ref[...] + jnp.dot(
        tx, w_ref[...], preferred_element_type=jnp.float32)


_comb1_tc = pl.pallas_call(
    _comb1_body,
    grid=(GRID,),
    in_specs=[
        pl.BlockSpec((NC, BN, D), lambda i: (0, i, 0)),
        pl.BlockSpec((BN, D), lambda i: (i, 0)),
        pl.BlockSpec((D, D), lambda i: (0, 0)),
    ],
    out_specs=[
        pl.BlockSpec((BN, D), lambda i: (i, 0)),
        pl.BlockSpec((BN, D), lambda i: (i, 0)),
    ],
    out_shape=[
        jax.ShapeDtypeStruct((N, D), jnp.float32),
        jax.ShapeDtypeStruct((N, D), jnp.float32),
    ],
)


def _comb2_body(p_ref, x0_ref, acc_ref, w2_ref, b_ref, wn_ref, h_ref, hacc_ref):
    tx2 = 2.0 * (p_ref[0] + p_ref[1]) - x0_ref[...]
    h = acc_ref[...] + jnp.dot(tx2, w2_ref[...],
                               preferred_element_type=jnp.float32) + b_ref[...]
    h = jnp.maximum(h, 0.0)
    h_ref[...] = h
    hacc_ref[...] = jnp.dot(h, wn_ref[...], preferred_element_type=jnp.float32)


_comb2_tc = pl.pallas_call(
    _comb2_body,
    grid=(GRID,),
    in_specs=[
        pl.BlockSpec((NC, BN, D), lambda i: (0, i, 0)),
        pl.BlockSpec((BN, D), lambda i: (i, 0)),
        pl.BlockSpec((BN, D), lambda i: (i, 0)),
        pl.BlockSpec((D, D), lambda i: (0, 0)),
        pl.BlockSpec((1, D), lambda i: (0, 0)),
        pl.BlockSpec((D, D), lambda i: (0, 0)),
    ],
    out_specs=[
        pl.BlockSpec((BN, D), lambda i: (i, 0)),
        pl.BlockSpec((BN, D), lambda i: (i, 0)),
    ],
    out_shape=[
        jax.ShapeDtypeStruct((N, D), jnp.float32),
        jax.ShapeDtypeStruct((N, D), jnp.float32),
    ],
)


def _final_body(p_ref, x0_ref, acc_ref, w_ref, b_ref, o_ref):
    tx2 = 2.0 * (p_ref[0] + p_ref[1]) - x0_ref[...]
    o = acc_ref[...] + jnp.dot(tx2, w_ref[...],
                               preferred_element_type=jnp.float32) + b_ref[...]
    m = jnp.max(o, axis=1, keepdims=True)
    lse = m + jnp.log(jnp.sum(jnp.exp(o - m), axis=1, keepdims=True))
    o_ref[...] = o - lse


_final_tc = pl.pallas_call(
    _final_body,
    grid=(GRID,),
    in_specs=[
        pl.BlockSpec((NC, BN, D), lambda i: (0, i, 0)),
        pl.BlockSpec((BN, D), lambda i: (i, 0)),
        pl.BlockSpec((BN, D), lambda i: (i, 0)),
        pl.BlockSpec((D, D), lambda i: (0, 0)),
        pl.BlockSpec((1, D), lambda i: (0, 0)),
    ],
    out_specs=pl.BlockSpec((BN, D), lambda i: (i, 0)),
    out_shape=jax.ShapeDtypeStruct((N, D), jnp.float32),
)


def kernel(x, edge_index, edge_weight, W1, b1, W2, b2):
    pad = E_PAD - E
    src = jnp.concatenate(
        [edge_index[0], jnp.zeros((pad,), jnp.int32)]).reshape(NW, NB, 128)
    dst = jnp.concatenate(
        [edge_index[1], jnp.zeros((pad,), jnp.int32)]).reshape(NW, NB, 128)
    w = jnp.concatenate(
        [edge_weight, jnp.zeros((pad,), jnp.float32)]).reshape(NW, NB, 128)
    b1r = b1.reshape(1, D)
    b2r = b2.reshape(1, D)

    deg_parts = _deg_kernel(src, dst, w)
    dis = _dis_tc(deg_parts)
    nrm = _norm_kernel(src, dst, w, dis.reshape(NPAD))

    srcf = src.reshape(TB, 128)
    dstf = dst.reshape(TB, 128)
    nrmf = nrm.reshape(TB, 128)

    acc0 = _mm_tc(x, W1[0])
    p1 = _prop_kernel(x, srcf, dstf, nrmf)
    tx1, acc1 = _comb1_tc(p1, acc0, W1[1])
    p2 = _prop_kernel(tx1, srcf, dstf, nrmf)
    h, hacc = _comb2_tc(p2, x, acc1, W1[2], b1r, W2[0])
    p3 = _prop_kernel(h, srcf, dstf, nrmf)
    t1, acc2 = _comb1_tc(p3, hacc, W2[1])
    p4 = _prop_kernel(t1, srcf, dstf, nrmf)
    return _final_tc(p4, h, acc2, W2[2], b2r)
